# baseline (speedup 1.0000x reference)
import functools

import jax
import jax.numpy as jnp
from jax import lax
from jax.experimental import pallas as pl
from jax.experimental.pallas import tpu as pltpu
from jax.experimental.pallas import tpu_sc as plsc

_B = 4
_N = 2048
_NQ = _B * _N
_L = 16
_NC = 3 * (_N // _L - 1) + _L
_CB = 400


def _ladder(m1, m2, m3, d):
    t = jnp.maximum(m1, d)
    m1 = jnp.minimum(m1, d)
    t2 = jnp.maximum(m2, t)
    m2 = jnp.minimum(m2, t)
    m3 = jnp.minimum(m3, t2)
    return m1, m2, m3


def _sc_knn(p1t, p2t, c1t, out, kx, ky, kz, cr, cg, cb, qx, qy, qz,
            cd0, ci0, cd1, ci1, ovr, ovg, ovb):
    nw = 32
    qpw = _NQ // nw
    wid = lax.axis_index("c") * 16 + lax.axis_index("s")
    b = wid // (_N // qpw)
    qoff = (wid % (_N // qpw)) * qpw

    pltpu.sync_copy(p1t.at[pl.ds((b * 3 + 0) * _N, _N)], kx)
    pltpu.sync_copy(p1t.at[pl.ds((b * 3 + 1) * _N, _N)], ky)
    pltpu.sync_copy(p1t.at[pl.ds((b * 3 + 2) * _N, _N)], kz)
    pltpu.sync_copy(c1t.at[pl.ds((b * 3 + 0) * _N, _N)], cr)
    pltpu.sync_copy(c1t.at[pl.ds((b * 3 + 1) * _N, _N)], cg)
    pltpu.sync_copy(c1t.at[pl.ds((b * 3 + 2) * _N, _N)], cb)
    pltpu.sync_copy(p2t.at[pl.ds((b * 3 + 0) * _N + qoff, qpw)], qx)
    pltpu.sync_copy(p2t.at[pl.ds((b * 3 + 1) * _N + qoff, qpw)], qy)
    pltpu.sync_copy(p2t.at[pl.ds((b * 3 + 2) * _N + qoff, qpw)], qz)

    inf16 = jnp.full((_L,), jnp.inf, jnp.float32)
    zero16 = jnp.zeros((_L,), jnp.float32)
    one16 = jnp.ones((_L,), jnp.float32)
    izero16 = jnp.zeros((_L,), jnp.int32)

    lane = lax.iota(jnp.int32, _L)
    ninf16 = jnp.full((_L,), -jnp.inf, jnp.float32)

    _dn = lax.GatherDimensionNumbers(
        offset_dims=(), collapsed_slice_dims=(0,), start_index_map=(0,))

    def _shuf(v, idx):
        return lax.gather(v, idx[:, None], _dn, (1,),
                          mode=lax.GatherScatterMode.PROMISE_IN_BOUNDS)

    def _bfly(v, op):
        for s in (8, 4, 2, 1):
            v = op(v, _shuf(v, lane ^ s))
        return v

    def _qcoord(qv, g, sel):
        return _bfly(jnp.where(sel, qv[pl.ds(g * _L, _L)], ninf16),
                     jnp.maximum)

    def pair_body(jp, _):
        j0 = jp * 2
        g0 = j0 // _L
        sel0 = lane == (j0 % _L)
        sel1 = lane == (j0 % _L + 1)
        qx0 = _qcoord(qx, g0, sel0)
        qy0 = _qcoord(qy, g0, sel0)
        qz0 = _qcoord(qz, g0, sel0)
        qx1 = _qcoord(qx, g0, sel1)
        qy1 = _qcoord(qy, g0, sel1)
        qz1 = _qcoord(qz, g0, sel1)

        cd0[pl.ds(_NC - 13, _L)] = inf16
        ci0[pl.ds(_NC - 13, _L)] = izero16
        cd1[pl.ds(_NC - 13, _L)] = inf16
        ci1[pl.ds(_NC - 13, _L)] = izero16

        def p1_body(c, _):
            off = c * _L
            kxc = kx[pl.ds(off, _L)]
            kyc = ky[pl.ds(off, _L)]
            kzc = kz[pl.ds(off, _L)]
            vidx = lane + off
            dxa = kxc - qx0
            dya = kyc - qy0
            dza = kzc - qz0
            da = (dxa * dxa + dya * dya) + dza * dza
            dxb = kxc - qx1
            dyb = kyc - qy1
            dzb = kzc - qz1
            db = (dxb * dxb + dyb * dyb) + dzb * dzb
            ska, sva = plsc.sort_key_val(da, vidx)
            skb, svb = plsc.sort_key_val(db, vidx)
            cd0[pl.ds(3 * c, _L)] = ska
            ci0[pl.ds(3 * c, _L)] = sva
            cd1[pl.ds(3 * c, _L)] = skb
            ci1[pl.ds(3 * c, _L)] = svb
            return 0

        lax.fori_loop(0, _N // _L, p1_body, 0)

        def finish(cd, ci, sel):
            def pc_body(i, ms):
                m1a, m2a, m3a, m1b, m2b, m3b = ms
                m1a, m2a, m3a = _ladder(m1a, m2a, m3a, cd[pl.ds(i * 32, _L)])
                m1b, m2b, m3b = _ladder(m1b, m2b, m3b,
                                        cd[pl.ds(i * 32 + _L, _L)])
                return (m1a, m2a, m3a, m1b, m2b, m3b)

            m1, m2, m3, m1b, m2b, m3b = lax.fori_loop(
                0, _CB // 32, pc_body, (inf16,) * 6)
            m1, m2, m3 = _ladder(m1, m2, m3, cd[pl.ds(_CB - _L, _L)])
            for v in (m1b, m2b, m3b):
                m1, m2, m3 = _ladder(m1, m2, m3, v)
            r1 = _bfly(m1, jnp.minimum)
            e1 = m1 == r1
            m1 = jnp.where(e1, m2, m1)
            m2 = jnp.where(e1, m3, m2)
            r2 = _bfly(m1, jnp.minimum)
            e2 = m1 == r2
            m1 = jnp.where(e2, m2, m1)
            thr = _bfly(m1, jnp.minimum)

            def p2_body(i, acc):
                ar, ag, ab, cn = acc
                cdv = cd[pl.ds(i * _L, _L)]
                civ = ci[pl.ds(i * _L, _L)]
                sa = cdv <= thr
                gr = plsc.load_gather(cr, [civ])
                gg = plsc.load_gather(cg, [civ])
                gb = plsc.load_gather(cb, [civ])
                ar = ar + jnp.where(sa, gr, zero16)
                ag = ag + jnp.where(sa, gg, zero16)
                ab = ab + jnp.where(sa, gb, zero16)
                cn = cn + jnp.where(sa, one16, zero16)
                return (ar, ag, ab, cn)

            ar, ag, ab, cn = lax.fori_loop(
                0, _CB // _L, p2_body, (zero16,) * 4)
            inv = one16 / _bfly(cn, jnp.add)
            g = g0
            ovr[pl.ds(g * _L, _L)] = jnp.where(sel, _bfly(ar, jnp.add) * inv,
                                               ovr[pl.ds(g * _L, _L)])
            ovg[pl.ds(g * _L, _L)] = jnp.where(sel, _bfly(ag, jnp.add) * inv,
                                               ovg[pl.ds(g * _L, _L)])
            ovb[pl.ds(g * _L, _L)] = jnp.where(sel, _bfly(ab, jnp.add) * inv,
                                               ovb[pl.ds(g * _L, _L)])

        finish(cd0, ci0, sel0)
        finish(cd1, ci1, sel1)
        return 0

    lax.fori_loop(0, qpw // 2, pair_body, 0)

    base = b * _N + qoff
    pltpu.sync_copy(ovr, out.at[pl.ds(0 * _NQ + base, qpw)])
    pltpu.sync_copy(ovg, out.at[pl.ds(1 * _NQ + base, qpw)])
    pltpu.sync_copy(ovb, out.at[pl.ds(2 * _NQ + base, qpw)])


def kernel(points1, points2, colors1):
    f32 = jnp.float32
    i32 = jnp.int32
    p1t = jnp.transpose(points1, (0, 2, 1)).reshape(_B * 3 * _N)
    p2t = jnp.transpose(points2, (0, 2, 1)).reshape(_B * 3 * _N)
    c1t = jnp.transpose(colors1, (0, 2, 1)).reshape(_B * 3 * _N)

    mesh = plsc.VectorSubcoreMesh(core_axis_name="c", subcore_axis_name="s")
    sc = functools.partial(
        pl.kernel,
        mesh=mesh,
        compiler_params=pltpu.CompilerParams(needs_layout_passes=False),
        out_type=jax.ShapeDtypeStruct((3 * _NQ,), f32),
        scratch_types=[
            pltpu.VMEM((_N,), f32),
            pltpu.VMEM((_N,), f32),
            pltpu.VMEM((_N,), f32),
            pltpu.VMEM((_N,), f32),
            pltpu.VMEM((_N,), f32),
            pltpu.VMEM((_N,), f32),
            pltpu.VMEM((_NQ // 32,), f32),
            pltpu.VMEM((_NQ // 32,), f32),
            pltpu.VMEM((_NQ // 32,), f32),
            pltpu.VMEM((_CB,), f32),
            pltpu.VMEM((_CB,), i32),
            pltpu.VMEM((_CB,), f32),
            pltpu.VMEM((_CB,), i32),
            pltpu.VMEM((_NQ // 32,), f32),
            pltpu.VMEM((_NQ // 32,), f32),
            pltpu.VMEM((_NQ // 32,), f32),
        ],
    )(_sc_knn)
    out_t = sc(p1t, p2t, c1t)
    return jnp.transpose(out_t.reshape(3, _B, _N), (1, 2, 0))


# speedup vs baseline: 2.2791x; 2.2791x over previous
import functools

import jax
import jax.numpy as jnp
from jax import lax
from jax.experimental import pallas as pl
from jax.experimental.pallas import tpu as pltpu
from jax.experimental.pallas import tpu_sc as plsc

_B = 4
_N = 2048
_NQ = _B * _N
_L = 16


def _ladder(m1, m2, m3, d):
    t = jnp.maximum(m1, d)
    m1 = jnp.minimum(m1, d)
    t2 = jnp.maximum(m2, t)
    m2 = jnp.minimum(m2, t)
    m3 = jnp.minimum(m3, t2)
    return m1, m2, m3


def _sc_knn(p1t, p2t, c1t, out, kx, ky, kz, cr, cg, cb, qx, qy, qz,
            db0, db1, ovr, ovg, ovb):
    nw = 32
    qpw = _NQ // nw
    wid = lax.axis_index("c") * 16 + lax.axis_index("s")
    b = wid // (_N // qpw)
    qoff = (wid % (_N // qpw)) * qpw

    pltpu.sync_copy(p1t.at[pl.ds((b * 3 + 0) * _N, _N)], kx)
    pltpu.sync_copy(p1t.at[pl.ds((b * 3 + 1) * _N, _N)], ky)
    pltpu.sync_copy(p1t.at[pl.ds((b * 3 + 2) * _N, _N)], kz)
    pltpu.sync_copy(c1t.at[pl.ds((b * 3 + 0) * _N, _N)], cr)
    pltpu.sync_copy(c1t.at[pl.ds((b * 3 + 1) * _N, _N)], cg)
    pltpu.sync_copy(c1t.at[pl.ds((b * 3 + 2) * _N, _N)], cb)
    pltpu.sync_copy(p2t.at[pl.ds((b * 3 + 0) * _N + qoff, qpw)], qx)
    pltpu.sync_copy(p2t.at[pl.ds((b * 3 + 1) * _N + qoff, qpw)], qy)
    pltpu.sync_copy(p2t.at[pl.ds((b * 3 + 2) * _N + qoff, qpw)], qz)

    inf16 = jnp.full((_L,), jnp.inf, jnp.float32)
    zero16 = jnp.zeros((_L,), jnp.float32)
    one16 = jnp.ones((_L,), jnp.float32)

    lane = lax.iota(jnp.int32, _L)
    ninf16 = jnp.full((_L,), -jnp.inf, jnp.float32)

    _dn = lax.GatherDimensionNumbers(
        offset_dims=(), collapsed_slice_dims=(0,), start_index_map=(0,))

    def _shuf(v, idx):
        return lax.gather(v, idx[:, None], _dn, (1,),
                          mode=lax.GatherScatterMode.PROMISE_IN_BOUNDS)

    def _bfly(v, op):
        for s in (8, 4, 2, 1):
            v = op(v, _shuf(v, lane ^ s))
        return v

    def _qcoord(qv, g, sel):
        return _bfly(jnp.where(sel, qv[pl.ds(g * _L, _L)], ninf16),
                     jnp.maximum)

    def _thresh(m1, m2, m3):
        r1 = _bfly(m1, jnp.minimum)
        e1 = m1 == r1
        m1 = jnp.where(e1, m2, m1)
        m2 = jnp.where(e1, m3, m2)
        r2 = _bfly(m1, jnp.minimum)
        e2 = m1 == r2
        m1 = jnp.where(e2, m2, m1)
        return _bfly(m1, jnp.minimum)

    def pair_body(jp, _):
        j0 = jp * 2
        g = j0 // _L
        sel0 = lane == (j0 % _L)
        sel1 = lane == (j0 % _L + 1)
        qx0 = _qcoord(qx, g, sel0)
        qy0 = _qcoord(qy, g, sel0)
        qz0 = _qcoord(qz, g, sel0)
        qx1 = _qcoord(qx, g, sel1)
        qy1 = _qcoord(qy, g, sel1)
        qz1 = _qcoord(qz, g, sel1)

        def p1_body(c, ms):
            m10, m20, m30, m11, m21, m31 = ms
            off = c * _L
            kxc = kx[pl.ds(off, _L)]
            kyc = ky[pl.ds(off, _L)]
            kzc = kz[pl.ds(off, _L)]
            dxa = kxc - qx0
            dya = kyc - qy0
            dza = kzc - qz0
            da = (dxa * dxa + dya * dya) + dza * dza
            db0[pl.ds(off, _L)] = da
            dxb = kxc - qx1
            dyb = kyc - qy1
            dzb = kzc - qz1
            dbv = (dxb * dxb + dyb * dyb) + dzb * dzb
            db1[pl.ds(off, _L)] = dbv
            m10, m20, m30 = _ladder(m10, m20, m30, da)
            m11, m21, m31 = _ladder(m11, m21, m31, dbv)
            return (m10, m20, m30, m11, m21, m31)

        m10, m20, m30, m11, m21, m31 = lax.fori_loop(
            0, _N // _L, p1_body, (inf16,) * 6)
        thr0 = _thresh(m10, m20, m30)
        thr1 = _thresh(m11, m21, m31)

        def p2_body(c, acc):
            ar0, ag0, ab0, cn0, ar1, ag1, ab1, cn1 = acc
            off = c * _L
            crc = cr[pl.ds(off, _L)]
            cgc = cg[pl.ds(off, _L)]
            cbc = cb[pl.ds(off, _L)]
            d0 = db0[pl.ds(off, _L)]
            s0 = d0 <= thr0
            ar0 = ar0 + jnp.where(s0, crc, zero16)
            ag0 = ag0 + jnp.where(s0, cgc, zero16)
            ab0 = ab0 + jnp.where(s0, cbc, zero16)
            cn0 = cn0 + plsc.all_reduce_population_count(s0)
            d1 = db1[pl.ds(off, _L)]
            s1 = d1 <= thr1
            ar1 = ar1 + jnp.where(s1, crc, zero16)
            ag1 = ag1 + jnp.where(s1, cgc, zero16)
            ab1 = ab1 + jnp.where(s1, cbc, zero16)
            cn1 = cn1 + plsc.all_reduce_population_count(s1)
            return (ar0, ag0, ab0, cn0, ar1, ag1, ab1, cn1)

        izero = jnp.zeros((_L,), jnp.int32)
        ar0, ag0, ab0, cn0, ar1, ag1, ab1, cn1 = lax.fori_loop(
            0, _N // _L, p2_body,
            (zero16, zero16, zero16, izero, zero16, zero16, zero16, izero))

        inv0 = one16 / cn0.astype(jnp.float32)
        ovr[pl.ds(g * _L, _L)] = jnp.where(sel0, _bfly(ar0, jnp.add) * inv0,
                                           ovr[pl.ds(g * _L, _L)])
        ovg[pl.ds(g * _L, _L)] = jnp.where(sel0, _bfly(ag0, jnp.add) * inv0,
                                           ovg[pl.ds(g * _L, _L)])
        ovb[pl.ds(g * _L, _L)] = jnp.where(sel0, _bfly(ab0, jnp.add) * inv0,
                                           ovb[pl.ds(g * _L, _L)])
        inv1 = one16 / cn1.astype(jnp.float32)
        ovr[pl.ds(g * _L, _L)] = jnp.where(sel1, _bfly(ar1, jnp.add) * inv1,
                                           ovr[pl.ds(g * _L, _L)])
        ovg[pl.ds(g * _L, _L)] = jnp.where(sel1, _bfly(ag1, jnp.add) * inv1,
                                           ovg[pl.ds(g * _L, _L)])
        ovb[pl.ds(g * _L, _L)] = jnp.where(sel1, _bfly(ab1, jnp.add) * inv1,
                                           ovb[pl.ds(g * _L, _L)])
        return 0

    lax.fori_loop(0, qpw // 2, pair_body, 0)

    base = b * _N + qoff
    pltpu.sync_copy(ovr, out.at[pl.ds(0 * _NQ + base, qpw)])
    pltpu.sync_copy(ovg, out.at[pl.ds(1 * _NQ + base, qpw)])
    pltpu.sync_copy(ovb, out.at[pl.ds(2 * _NQ + base, qpw)])


def kernel(points1, points2, colors1):
    f32 = jnp.float32
    p1t = jnp.transpose(points1, (0, 2, 1)).reshape(_B * 3 * _N)
    p2t = jnp.transpose(points2, (0, 2, 1)).reshape(_B * 3 * _N)
    c1t = jnp.transpose(colors1, (0, 2, 1)).reshape(_B * 3 * _N)

    mesh = plsc.VectorSubcoreMesh(core_axis_name="c", subcore_axis_name="s")
    sc = functools.partial(
        pl.kernel,
        mesh=mesh,
        compiler_params=pltpu.CompilerParams(needs_layout_passes=False),
        out_type=jax.ShapeDtypeStruct((3 * _NQ,), f32),
        scratch_types=[
            pltpu.VMEM((_N,), f32),
            pltpu.VMEM((_N,), f32),
            pltpu.VMEM((_N,), f32),
            pltpu.VMEM((_N,), f32),
            pltpu.VMEM((_N,), f32),
            pltpu.VMEM((_N,), f32),
            pltpu.VMEM((_NQ // 32,), f32),
            pltpu.VMEM((_NQ // 32,), f32),
            pltpu.VMEM((_NQ // 32,), f32),
            pltpu.VMEM((_N,), f32),
            pltpu.VMEM((_N,), f32),
            pltpu.VMEM((_NQ // 32,), f32),
            pltpu.VMEM((_NQ // 32,), f32),
            pltpu.VMEM((_NQ // 32,), f32),
        ],
    )(_sc_knn)
    out_t = sc(p1t, p2t, c1t)
    return jnp.transpose(out_t.reshape(3, _B, _N), (1, 2, 0))
